# baseline (speedup 1.0000x reference)
import functools

import jax
import jax.numpy as jnp
from jax import lax
from jax.experimental import pallas as pl
from jax.experimental.pallas import tpu as pltpu
from jax.experimental.pallas import tpu_sc as plsc

NX = 128
NY = 128
C = 80
CH = C // 2
NB = 10
NSEG = NX * NY
NPTS = 249216
KB = NPTS // 128
NT = 16
ROWS_PER_TILE = NSEG // NT


_mesh = plsc.VectorSubcoreMesh(core_axis_name="c", subcore_axis_name="s")


@functools.partial(
    pl.kernel,
    out_type=jax.ShapeDtypeStruct((2, NSEG, CH), jnp.float32),
    mesh=_mesh,
    compiler_params=pltpu.CompilerParams(
        use_tc_tiling_on_sc=False, needs_layout_passes=False),
    scratch_types=[
        pltpu.VMEM_SHARED((NSEG + 1, CH), jnp.float32),
        pltpu.VMEM((NB // 2, 8, 128), jnp.float32),
        pltpu.VMEM((NB // 2, 8, 128), jnp.float32),
        pltpu.VMEM((2, 128), jnp.int32),
        pltpu.VMEM((2, 128), jnp.int32),
        pltpu.VMEM((1, 128), jnp.int32),
        pltpu.VMEM((1, 128), jnp.int32),
        pltpu.VMEM((128, CH), jnp.float32),
        pltpu.VMEM((128, CH), jnp.float32),
        pltpu.VMEM((16, CH), jnp.float32),
        pltpu.SemaphoreType.DMA,
        pltpu.SemaphoreType.DMA,
        pltpu.SemaphoreType.DMA,
        pltpu.SemaphoreType.DMA,
    ],
)
def _sc_bev_scatter(p_hbm, g_hbm, out_hbm, accum, xn0, xn1, gbuf0, gbuf1,
                    ranks0, ranks1, xrow0, xrow1, zbuf, sx0, sx1, sg0, sg1):
    cid = lax.axis_index("c")
    sid = lax.axis_index("s")

    zeros16 = jnp.zeros((16,), jnp.float32)

    def _zrow(i, _):
        for k in range(CH // 16):
            zbuf[i, pl.ds(16 * k, 16)] = zeros16
        return 0

    lax.fori_loop(0, 16, _zrow, 0)
    row0 = sid * ROWS_PER_TILE

    def _zdma(j, _):
        pltpu.sync_copy(zbuf, accum.at[pl.ds(row0 + 16 * j, 16), :])
        return 0

    lax.fori_loop(0, ROWS_PER_TILE // 16, _zdma, 0)
    plsc.subcore_barrier()

    band0 = cid * (NB // 2)

    def _process(m, xn, gbuf, ranks, xrow, sx, sg):
        iota16 = lax.iota(jnp.int32, 16)
        idx_pts = [iota16 + 16 * g for g in range(8)]
        k_raw = sid + NT * m
        k = lax.rem(k_raw, KB)
        valid = k_raw < KB
        pltpu.sync_copy(p_hbm.at[pl.ds(band0, NB // 2), k], xn)
        pltpu.sync_copy(g_hbm.at[k], gbuf)

        for i in range(8):
            r = (gbuf[0, pl.ds(16 * i, 16)] * NY
                 + gbuf[1, pl.ds(16 * i, 16)])
            ranks[0, pl.ds(16 * i, 16)] = jnp.where(valid, r, NSEG)
        for b in range(NB // 2):
            for r8 in range(8):
                ch = jnp.full((16,), 8 * b + r8, jnp.int32)
                for g in range(8):
                    v = xn[b, r8, pl.ds(16 * g, 16)]
                    plsc.store_scatter(xrow, [idx_pts[g], ch], v)

        pltpu.sync_copy(xrow, accum.at[ranks.at[0]], add=True)

    def _pair(jp, _):
        m0 = 2 * jp
        _process(m0, xn0, gbuf0, ranks0, xrow0, sx0, sg0)
        _process(m0 + 1, xn1, gbuf1, ranks1, xrow1, sx1, sg1)
        return 0

    lax.fori_loop(0, (KB // NT + 2) // 2, _pair, 0)
    plsc.subcore_barrier()

    pltpu.sync_copy(
        accum.at[pl.ds(row0, ROWS_PER_TILE), :],
        out_hbm.at[cid, pl.ds(row0, ROWS_PER_TILE), :],
    )


def _tc_combine_body(p_ref, o_ref):
    o_ref[pl.ds(0, CH), :] = p_ref[0].T
    o_ref[pl.ds(CH, CH), :] = p_ref[1].T


_TC_BLK = 1024


def _tc_combine(partials):
    return pl.pallas_call(
        _tc_combine_body,
        grid=(NSEG // _TC_BLK,),
        in_specs=[pl.BlockSpec((2, _TC_BLK, CH), lambda j: (0, j, 0))],
        out_specs=pl.BlockSpec((C, _TC_BLK), lambda j: (0, j)),
        out_shape=jax.ShapeDtypeStruct((C, NSEG), jnp.float32),
    )(partials)


@jax.jit
def kernel(x, geom_xy):
    p = x.reshape(KB, 128, NB, 8).transpose(2, 0, 3, 1)
    g = geom_xy.reshape(KB, 128, 2).transpose(0, 2, 1)
    partials = _sc_bev_scatter(p, g)
    out = _tc_combine(partials)
    return out.reshape(1, C, NX, NY)


# speedup vs baseline: 1.1090x; 1.1090x over previous
import functools

import jax
import jax.numpy as jnp
from jax import lax
from jax.experimental import pallas as pl
from jax.experimental.pallas import tpu as pltpu
from jax.experimental.pallas import tpu_sc as plsc

NX = 128
NY = 128
C = 80
NSEG = NX * NY
NPTS = 249216
CHUNK = 384
NCHUNKS = NPTS // CHUNK
NW = 32
ROWS_PER_TILE = NSEG // 16


_mesh = plsc.VectorSubcoreMesh(core_axis_name="c", subcore_axis_name="s")


@functools.partial(
    pl.kernel,
    out_type=jax.ShapeDtypeStruct((2, NSEG, C), jnp.float32),
    mesh=_mesh,
    compiler_params=pltpu.CompilerParams(use_tc_tiling_on_sc=False),
    scratch_types=[
        pltpu.VMEM_SHARED((NSEG, C), jnp.float32),
        pltpu.VMEM((CHUNK, C), jnp.float32),
        pltpu.VMEM((CHUNK,), jnp.int32),
        pltpu.VMEM((CHUNK,), jnp.int32),
        pltpu.VMEM((CHUNK // 128, 128), jnp.int32),
        pltpu.VMEM((16, C), jnp.float32),
    ],
)
def _sc_bev_scatter(x_hbm, gx_hbm, gy_hbm, out_hbm, accum, xbuf, gxbuf, gybuf,
                    ranks, zbuf):
    cid_core = lax.axis_index("c")
    sid = lax.axis_index("s")
    wid = sid * 2 + cid_core

    zeros16 = jnp.zeros((16,), jnp.float32)

    def _zrow(i, _):
        for k in range(C // 16):
            zbuf[i, pl.ds(16 * k, 16)] = zeros16
        return 0

    lax.fori_loop(0, 16, _zrow, 0)
    row0 = sid * ROWS_PER_TILE

    def _zdma(j, _):
        pltpu.sync_copy(zbuf, accum.at[pl.ds(row0 + 16 * j, 16), :])
        return 0

    lax.fori_loop(0, ROWS_PER_TILE // 16, _zdma, 0)
    plsc.subcore_barrier()

    n_extra = NCHUNKS - NW * (NCHUNKS // NW)
    nj = jnp.where(wid < n_extra, NCHUNKS // NW + 1, NCHUNKS // NW)

    def _chunk(j, _):
        base = (wid + NW * j) * CHUNK
        pltpu.sync_copy(x_hbm.at[pl.ds(base, CHUNK), :], xbuf)
        pltpu.sync_copy(gx_hbm.at[pl.ds(base, CHUNK)], gxbuf)
        pltpu.sync_copy(gy_hbm.at[pl.ds(base, CHUNK)], gybuf)
        for sb in range(CHUNK // 128):
            for i in range(8):
                off = sb * 128 + i * 16
                r = gxbuf[pl.ds(off, 16)] * NY + gybuf[pl.ds(off, 16)]
                ranks[sb, pl.ds(i * 16, 16)] = r
        for sb in range(CHUNK // 128):
            pltpu.sync_copy(
                xbuf.at[pl.ds(sb * 128, 128), :],
                accum.at[ranks.at[sb]],
                add=True,
            )
        return 0

    lax.fori_loop(0, nj, _chunk, 0)
    plsc.subcore_barrier()

    pltpu.sync_copy(
        accum.at[pl.ds(row0, ROWS_PER_TILE), :],
        out_hbm.at[cid_core, pl.ds(row0, ROWS_PER_TILE), :],
    )


def _tc_combine_body(p_ref, o_ref):
    s = p_ref[0] + p_ref[1]
    o_ref[...] = s.T


_TC_BLK = 1024


def _tc_combine(partials):
    return pl.pallas_call(
        _tc_combine_body,
        grid=(NSEG // _TC_BLK,),
        in_specs=[pl.BlockSpec((2, _TC_BLK, C), lambda j: (0, j, 0))],
        out_specs=pl.BlockSpec((C, _TC_BLK), lambda j: (0, j)),
        out_shape=jax.ShapeDtypeStruct((C, NSEG), jnp.float32),
    )(partials)


@jax.jit
def kernel(x, geom_xy):
    gx = geom_xy[:, 0]
    gy = geom_xy[:, 1]
    partials = _sc_bev_scatter(x, gx, gy)
    out = _tc_combine(partials)
    return out.reshape(1, C, NX, NY)
